# baseline (speedup 1.0000x reference)
import jax
import jax.numpy as jnp
from jax.experimental import pallas as pl

B, S, D = 2, 4096, 1024
E, R = 64, 8
T = B * S
TB = 2048


def _moe_body(x_ref, wr_ref, wa_ref, wb_ref, col_ref, o_ref):
    x = x_ref[...]
    nrm = jnp.sqrt(jnp.sum(x * x, axis=1, keepdims=True))
    xn = x / jnp.maximum(nrm, 1e-12)
    logits = jax.lax.dot_general(
        xn, wr_ref[...], (((1,), (1,)), ((), ())),
        preferred_element_type=jnp.float32)
    mx = jnp.max(logits, axis=1, keepdims=True)
    colE = jax.lax.broadcasted_iota(jnp.int32, (TB, E), 1)
    eid = jnp.min(jnp.where(logits == mx, colE, E), axis=1, keepdims=True)

    h_all = jax.lax.dot_general(
        x.astype(jnp.bfloat16), wa_ref[...],
        (((1,), (1,)), ((), ())),
        preferred_element_type=jnp.float32)
    hexp = jnp.where(col_ref[...] == eid, h_all, 0.0)
    o_ref[...] = jax.lax.dot_general(
        hexp.astype(jnp.bfloat16), wb_ref[...],
        (((1,), (0,)), ((), ())),
        preferred_element_type=jnp.float32)


def kernel(hidden_states, Wr, Wa, Wb):
    b, s, d = hidden_states.shape
    x = hidden_states.reshape(-1, d)
    wa_flat = Wa.reshape(E * R, D).astype(jnp.bfloat16)
    wb_flat = Wb.transpose(0, 2, 1).reshape(E * R, D).astype(jnp.bfloat16)
    col = (jnp.arange(E * R, dtype=jnp.int32) // R).reshape(1, E * R)
    out = pl.pallas_call(
        _moe_body,
        grid=(T // TB,),
        in_specs=[
            pl.BlockSpec((TB, D), lambda i: (i, 0)),
            pl.BlockSpec((E, D), lambda i: (0, 0)),
            pl.BlockSpec((E * R, D), lambda i: (0, 0)),
            pl.BlockSpec((E * R, D), lambda i: (0, 0)),
            pl.BlockSpec((1, E * R), lambda i: (0, 0)),
        ],
        out_specs=pl.BlockSpec((TB, D), lambda i: (i, 0)),
        out_shape=jax.ShapeDtypeStruct((T, D), jnp.float32),
    )(x, Wr, wa_flat, wb_flat, col)
    return out.reshape(b, s, d)


# speedup vs baseline: 1.0498x; 1.0498x over previous
import jax
import jax.numpy as jnp
from jax.experimental import pallas as pl

B, S, D = 2, 4096, 1024
E, R = 64, 8
T = B * S
TB = 1024


def _moe_body(x_ref, wr_ref, wa_ref, wb_ref, col_ref, o_ref):
    x = x_ref[...]
    nrm = jnp.sqrt(jnp.sum(x * x, axis=1, keepdims=True))
    xn = x / jnp.maximum(nrm, 1e-12)
    logits = jax.lax.dot_general(
        xn, wr_ref[...], (((1,), (1,)), ((), ())),
        preferred_element_type=jnp.float32)
    mx = jnp.max(logits, axis=1, keepdims=True)
    colE = jax.lax.broadcasted_iota(jnp.int32, (TB, E), 1)
    eid = jnp.min(jnp.where(logits == mx, colE, E), axis=1, keepdims=True)

    h_all = jax.lax.dot_general(
        x.astype(jnp.bfloat16), wa_ref[...],
        (((1,), (1,)), ((), ())),
        preferred_element_type=jnp.float32)
    hexp = jnp.where(col_ref[...] == eid, h_all, 0.0)
    o_ref[...] = jax.lax.dot_general(
        hexp.astype(jnp.bfloat16), wb_ref[...],
        (((1,), (0,)), ((), ())),
        preferred_element_type=jnp.float32)


def kernel(hidden_states, Wr, Wa, Wb):
    b, s, d = hidden_states.shape
    x = hidden_states.reshape(-1, d)
    wa_flat = Wa.reshape(E * R, D).astype(jnp.bfloat16)
    wb_flat = Wb.transpose(0, 2, 1).reshape(E * R, D).astype(jnp.bfloat16)
    col = (jnp.arange(E * R, dtype=jnp.int32) // R).reshape(1, E * R)
    out = pl.pallas_call(
        _moe_body,
        grid=(T // TB,),
        in_specs=[
            pl.BlockSpec((TB, D), lambda i: (i, 0)),
            pl.BlockSpec((E, D), lambda i: (0, 0)),
            pl.BlockSpec((E * R, D), lambda i: (0, 0)),
            pl.BlockSpec((E * R, D), lambda i: (0, 0)),
            pl.BlockSpec((1, E * R), lambda i: (0, 0)),
        ],
        out_specs=pl.BlockSpec((TB, D), lambda i: (i, 0)),
        out_shape=jax.ShapeDtypeStruct((T, D), jnp.float32),
    )(x, Wr, wa_flat, wb_flat, col)
    return out.reshape(b, s, d)


# speedup vs baseline: 1.0705x; 1.0198x over previous
import jax
import jax.numpy as jnp
from jax.experimental import pallas as pl

B, S, D = 2, 4096, 1024
E, R = 64, 8
T = B * S
TB = 1024


def _moe_body(x_ref, wr_ref, wa_ref, wb_ref, o_ref):
    x = x_ref[...]
    nrm = jnp.sqrt(jnp.sum(x * x, axis=1, keepdims=True))
    xn = x / jnp.maximum(nrm, 1e-12)
    logits = jax.lax.dot_general(
        xn, wr_ref[...], (((1,), (1,)), ((), ())),
        preferred_element_type=jnp.float32)
    mx = jnp.max(logits, axis=1, keepdims=True)
    colE = jax.lax.broadcasted_iota(jnp.int32, (TB, E), 1)
    eid = jnp.min(jnp.where(logits == mx, colE, E), axis=1, keepdims=True)

    h_all = jax.lax.dot_general(
        x.astype(jnp.bfloat16), wa_ref[...],
        (((1,), (1,)), ((), ())),
        preferred_element_type=jnp.float32)
    col = jax.lax.broadcasted_iota(jnp.int32, (TB, E * R), 1) // R
    hexp = jnp.where(col == eid, h_all, 0.0)
    o_ref[...] = jax.lax.dot_general(
        hexp.astype(jnp.bfloat16), wb_ref[...],
        (((1,), (0,)), ((), ())),
        preferred_element_type=jnp.float32)


def kernel(hidden_states, Wr, Wa, Wb):
    b, s, d = hidden_states.shape
    x = hidden_states.reshape(-1, d)
    wa_flat = Wa.reshape(E * R, D).astype(jnp.bfloat16)
    wb_flat = Wb.transpose(0, 2, 1).reshape(E * R, D).astype(jnp.bfloat16)
    out = pl.pallas_call(
        _moe_body,
        grid=(T // TB,),
        in_specs=[
            pl.BlockSpec((TB, D), lambda i: (i, 0)),
            pl.BlockSpec((E, D), lambda i: (0, 0)),
            pl.BlockSpec((E * R, D), lambda i: (0, 0)),
            pl.BlockSpec((E * R, D), lambda i: (0, 0)),
        ],
        out_specs=pl.BlockSpec((TB, D), lambda i: (i, 0)),
        out_shape=jax.ShapeDtypeStruct((T, D), jnp.float32),
    )(x, Wr, wa_flat, wb_flat)
    return out.reshape(b, s, d)
